# baseline (speedup 1.0000x reference)
import jax
import jax.numpy as jnp
from jax.experimental import pallas as pl
from jax.experimental.pallas import tpu as pltpu
from jax.experimental.pallas import tpu_sc as plsc

QUEUE_SIZE = 131072
FEATURE_DIM = 128
BATCH = 16384

FBLK = 128
HEAD_STEPS = BATCH // FBLK
TAIL_STEPS = (QUEUE_SIZE - BATCH) // FBLK

LBL_COLS = 128
LBL_ROWS_Q = QUEUE_SIZE // LBL_COLS
LBL_ROWS_B = BATCH // LBL_COLS


def kernel(feat, true, pred, features, true_labels, pred_labels):
    true2d = true.reshape(LBL_ROWS_B, LBL_COLS)
    pred2d = pred.reshape(LBL_ROWS_B, LBL_COLS)
    tl2d = true_labels.reshape(LBL_ROWS_Q, LBL_COLS)
    pl2d = pred_labels.reshape(LBL_ROWS_Q, LBL_COLS)

    mesh = plsc.VectorSubcoreMesh(core_axis_name="c", subcore_axis_name="s")
    out_type = (
        jax.ShapeDtypeStruct((QUEUE_SIZE, FEATURE_DIM), jnp.float32),
        jax.ShapeDtypeStruct((LBL_ROWS_Q, LBL_COLS), jnp.int32),
        jax.ShapeDtypeStruct((LBL_ROWS_Q, LBL_COLS), jnp.int32),
    )

    @pl.kernel(out_type=out_type, mesh=mesh, scratch_types=[])
    def sc_kernel(feat_h, t_h, p_h, features_h, tl_h, pl_h,
                  of_h, ot_h, op_h):
        def body(f_in, t_in, p_in, f_out, t_out, p_out):
            f_out[...] = f_in[...]
            t_out[...] = t_in[...]
            p_out[...] = p_in[...]

        fblk = lambda: pl.BlockSpec((FBLK, FEATURE_DIM), lambda i: (i, 0))
        lblk = lambda: pl.BlockSpec((1, LBL_COLS), lambda i: (i, 0))
        pltpu.emit_pipeline(
            body,
            grid=(HEAD_STEPS,),
            in_specs=[fblk(), lblk(), lblk()],
            out_specs=[fblk(), lblk(), lblk()],
            core_axis_name=("c", "s"),
            dimension_semantics=(pltpu.PARALLEL,),
        )(feat_h, t_h, p_h, of_h, ot_h, op_h)

        ftblk = lambda: pl.BlockSpec((FBLK, FEATURE_DIM),
                                     lambda i: (i + HEAD_STEPS, 0))
        ltblk = lambda: pl.BlockSpec((1, LBL_COLS),
                                     lambda i: (i + HEAD_STEPS, 0))
        pltpu.emit_pipeline(
            body,
            grid=(TAIL_STEPS,),
            in_specs=[ftblk(), ltblk(), ltblk()],
            out_specs=[ftblk(), ltblk(), ltblk()],
            core_axis_name=("c", "s"),
            dimension_semantics=(pltpu.PARALLEL,),
        )(features_h, tl_h, pl_h, of_h, ot_h, op_h)

    out_f, out_t, out_p = sc_kernel(feat, true2d, pred2d, features, tl2d, pl2d)
    return (out_f, out_t.reshape(QUEUE_SIZE), out_p.reshape(QUEUE_SIZE))


# speedup vs baseline: 3.9241x; 3.9241x over previous
import jax
import jax.numpy as jnp
from jax.experimental import pallas as pl
from jax.experimental.pallas import tpu as pltpu

QUEUE_SIZE = 131072
FEATURE_DIM = 128
BATCH = 16384

BLOCK_ROWS = 8192
GRID = QUEUE_SIZE // BLOCK_ROWS
FEAT_BLOCKS = BATCH // BLOCK_ROWS

LBL_COLS = 128
LBL_ROWS_Q = QUEUE_SIZE // LBL_COLS
LBL_ROWS_B = BATCH // LBL_COLS
LBL_BLOCK = BLOCK_ROWS // LBL_COLS


def _copy_kernel(features, feat, true2d, pred2d, tl2d, pl2d,
                 out_f, out_t, out_p):
    i = pl.program_id(0)

    @pl.when(i < FEAT_BLOCKS)
    def _():
        out_f[...] = feat[...]
        out_t[...] = true2d[...]
        out_p[...] = pred2d[...]

    @pl.when(i >= FEAT_BLOCKS)
    def _():
        out_f[...] = features[...]
        out_t[...] = tl2d[...]
        out_p[...] = pl2d[...]


def kernel(feat, true, pred, features, true_labels, pred_labels):
    true2d = true.reshape(LBL_ROWS_B, LBL_COLS)
    pred2d = pred.reshape(LBL_ROWS_B, LBL_COLS)
    tl2d = true_labels.reshape(LBL_ROWS_Q, LBL_COLS)
    pl2d = pred_labels.reshape(LBL_ROWS_Q, LBL_COLS)

    new_idx = lambda i: (jnp.minimum(i, FEAT_BLOCKS - 1), 0)
    old_idx = lambda i: (jnp.maximum(i, FEAT_BLOCKS), 0)

    out_f, out_t, out_p = pl.pallas_call(
        _copy_kernel,
        grid=(GRID,),
        in_specs=[
            pl.BlockSpec((BLOCK_ROWS, FEATURE_DIM), old_idx),
            pl.BlockSpec((BLOCK_ROWS, FEATURE_DIM), new_idx),
            pl.BlockSpec((LBL_BLOCK, LBL_COLS), new_idx),
            pl.BlockSpec((LBL_BLOCK, LBL_COLS), new_idx),
            pl.BlockSpec((LBL_BLOCK, LBL_COLS), old_idx),
            pl.BlockSpec((LBL_BLOCK, LBL_COLS), old_idx),
        ],
        out_specs=[
            pl.BlockSpec((BLOCK_ROWS, FEATURE_DIM), lambda i: (i, 0)),
            pl.BlockSpec((LBL_BLOCK, LBL_COLS), lambda i: (i, 0)),
            pl.BlockSpec((LBL_BLOCK, LBL_COLS), lambda i: (i, 0)),
        ],
        out_shape=[
            jax.ShapeDtypeStruct((QUEUE_SIZE, FEATURE_DIM), jnp.float32),
            jax.ShapeDtypeStruct((LBL_ROWS_Q, LBL_COLS), jnp.int32),
            jax.ShapeDtypeStruct((LBL_ROWS_Q, LBL_COLS), jnp.int32),
        ],
        compiler_params=pltpu.CompilerParams(
            dimension_semantics=("arbitrary",),
        ),
    )(features, feat, true2d, pred2d, tl2d, pl2d)

    return (out_f, out_t.reshape(QUEUE_SIZE), out_p.reshape(QUEUE_SIZE))
